# baseline (speedup 1.0000x reference)
import functools

import jax
import jax.numpy as jnp
from jax.experimental import pallas as pl
from jax.experimental.pallas import tpu as pltpu


def _dot(a, b):
    return jnp.dot(a, b, preferred_element_type=jnp.float32)


def _fused_body(x_ref, *refs, ndim, inter):
    wr1, br1, wr2, br2, wr3, br3, wr4, br4 = refs[8 * ndim:8 * ndim + 8]
    out_ref, w4s, bc = refs[8 * ndim + 8:]

    b4sum = None
    for d in range(ndim):
        w4, b4 = refs[8 * d + 6], refs[8 * d + 7]
        w4s[d * inter:(d + 1) * inter, :] = _dot(w4[...], wr1[...])
        b4sum = b4[...] if b4sum is None else b4sum + b4[...]
    bc[...] = _dot(b4sum[None, :], wr1[...]) + br1[...][None, :]

    x = x_ref[...]
    acc = None
    for d in range(ndim):
        w1, b1, w2, b2, w3, b3 = refs[8 * d:8 * d + 6]
        col = x[:, d:d + 1]
        h = jnp.maximum(col * w1[...] + b1[...], 0.0)
        h = jnp.maximum(_dot(h, w2[...]) + b2[...], 0.0)
        h = jnp.maximum(_dot(h, w3[...]) + b3[...], 0.0)
        g = _dot(h, w4s[d * inter:(d + 1) * inter, :])
        acc = g if acc is None else acc + g
    h = jnp.maximum(acc + bc[...], 0.0)
    h = jnp.maximum(_dot(h, wr2[...]) + br2[...], 0.0)
    h = jnp.maximum(_dot(h, wr3[...]) + br3[...], 0.0)
    out_ref[...] = _dot(h, wr4[...]) + br4[...]


def kernel(x, nets, rho_params):
    n, ndim = x.shape
    inter = nets[0][-1][0].shape[1]
    zdim = rho_params[-1][0].shape[1]

    args = []
    for net in nets:
        for (w, b) in net:
            args += [w, b]
    for (w, b) in rho_params:
        args += [w, b]

    blk = 4096
    n_pad = ((n + blk - 1) // blk) * blk
    xp = x if n_pad == n else jnp.pad(x, ((0, n_pad - n), (0, 0)))

    full = lambda a: pl.BlockSpec(a.shape, lambda i: (0,) * a.ndim)
    out = pl.pallas_call(
        functools.partial(_fused_body, ndim=ndim, inter=inter),
        grid=(n_pad // blk,),
        in_specs=[pl.BlockSpec((blk, ndim), lambda i: (i, 0))]
                 + [full(a) for a in args],
        out_specs=pl.BlockSpec((blk, zdim), lambda i: (i, 0)),
        out_shape=jax.ShapeDtypeStruct((n_pad, zdim), jnp.float32),
        scratch_shapes=[pltpu.VMEM((ndim * inter, inter), jnp.float32),
                        pltpu.VMEM((1, inter), jnp.float32)],
        compiler_params=pltpu.CompilerParams(
            dimension_semantics=("parallel",)),
    )(xp, *args)
    return out[:n] if n_pad != n else out


# speedup vs baseline: 1.0007x; 1.0007x over previous
import functools

import jax
import jax.numpy as jnp
from jax.experimental import pallas as pl
from jax.experimental.pallas import tpu as pltpu


def _dot(a, b):
    return jnp.dot(a, b, preferred_element_type=jnp.float32)


def _fused_body(x_ref, *refs, ndim, inter):
    wr1, br1, wr2, br2, wr3, br3, wr4, br4 = refs[8 * ndim:8 * ndim + 8]
    out_ref, w4s, bc = refs[8 * ndim + 8:]

    @pl.when(pl.program_id(0) == 0)
    def _compose():
        b4sum = None
        for d in range(ndim):
            w4, b4 = refs[8 * d + 6], refs[8 * d + 7]
            w4s[d * inter:(d + 1) * inter, :] = _dot(w4[...], wr1[...])
            b4sum = b4[...] if b4sum is None else b4sum + b4[...]
        bc[...] = _dot(b4sum[None, :], wr1[...]) + br1[...][None, :]

    x = x_ref[...]
    acc = None
    for d in range(ndim):
        w1, b1, w2, b2, w3, b3 = refs[8 * d:8 * d + 6]
        col = x[:, d:d + 1]
        h = jnp.maximum(col * w1[...] + b1[...], 0.0)
        h = jnp.maximum(_dot(h, w2[...]) + b2[...], 0.0)
        h = jnp.maximum(_dot(h, w3[...]) + b3[...], 0.0)
        g = _dot(h, w4s[d * inter:(d + 1) * inter, :])
        acc = g if acc is None else acc + g
    h = jnp.maximum(acc + bc[...], 0.0)
    h = jnp.maximum(_dot(h, wr2[...]) + br2[...], 0.0)
    h = jnp.maximum(_dot(h, wr3[...]) + br3[...], 0.0)
    out_ref[...] = _dot(h, wr4[...]) + br4[...]


def kernel(x, nets, rho_params):
    n, ndim = x.shape
    inter = nets[0][-1][0].shape[1]
    zdim = rho_params[-1][0].shape[1]

    args = []
    for net in nets:
        for (w, b) in net:
            args += [w, b]
    for (w, b) in rho_params:
        args += [w, b]

    blk = 8192
    n_pad = ((n + blk - 1) // blk) * blk
    xp = x if n_pad == n else jnp.pad(x, ((0, n_pad - n), (0, 0)))

    full = lambda a: pl.BlockSpec(a.shape, lambda i: (0,) * a.ndim)
    out = pl.pallas_call(
        functools.partial(_fused_body, ndim=ndim, inter=inter),
        grid=(n_pad // blk,),
        in_specs=[pl.BlockSpec((blk, ndim), lambda i: (i, 0))]
                 + [full(a) for a in args],
        out_specs=pl.BlockSpec((blk, zdim), lambda i: (i, 0)),
        out_shape=jax.ShapeDtypeStruct((n_pad, zdim), jnp.float32),
        scratch_shapes=[pltpu.VMEM((ndim * inter, inter), jnp.float32),
                        pltpu.VMEM((1, inter), jnp.float32)],
    )(xp, *args)
    return out[:n] if n_pad != n else out


# speedup vs baseline: 1.0090x; 1.0083x over previous
import functools

import jax
import jax.numpy as jnp
from jax.experimental import pallas as pl
from jax.experimental.pallas import tpu as pltpu


def _dot(a, b):
    return jnp.dot(a, b, preferred_element_type=jnp.float32)


def _fused_body(x_ref, *refs, ndim, inter):
    wr1, br1, wr2, br2, wr3, br3, wr4, br4 = refs[8 * ndim:8 * ndim + 8]
    out_ref, w4s, bc = refs[8 * ndim + 8:]

    @pl.when(pl.program_id(0) == 0)
    def _compose():
        b4sum = None
        for d in range(ndim):
            w4, b4 = refs[8 * d + 6], refs[8 * d + 7]
            w4s[d * inter:(d + 1) * inter, :] = _dot(w4[...], wr1[...])
            b4sum = b4[...] if b4sum is None else b4sum + b4[...]
        bc[...] = _dot(b4sum[None, :], wr1[...]) + br1[...][None, :]

    x = x_ref[...]
    acc = None
    for d in range(ndim):
        w1, b1, w2, b2, w3, b3 = refs[8 * d:8 * d + 6]
        col = x[:, d:d + 1]
        h = jnp.maximum(col * w1[...] + b1[...], 0.0)
        h = jnp.maximum(_dot(h, w2[...]) + b2[...], 0.0)
        h = jnp.maximum(_dot(h, w3[...]) + b3[...], 0.0)
        g = _dot(h, w4s[d * inter:(d + 1) * inter, :])
        acc = g if acc is None else acc + g
    h = jnp.maximum(acc + bc[...], 0.0)
    h = jnp.maximum(_dot(h, wr2[...]) + br2[...], 0.0)
    h = jnp.maximum(_dot(h, wr3[...]) + br3[...], 0.0)
    out_ref[...] = _dot(h, wr4[...]) + br4[...]


def kernel(x, nets, rho_params):
    n, ndim = x.shape
    inter = nets[0][-1][0].shape[1]
    zdim = rho_params[-1][0].shape[1]

    args = []
    for net in nets:
        for (w, b) in net:
            args += [w, b]
    for (w, b) in rho_params:
        args += [w, b]

    blk = 4096
    n_pad = ((n + blk - 1) // blk) * blk
    xp = x if n_pad == n else jnp.pad(x, ((0, n_pad - n), (0, 0)))

    full = lambda a: pl.BlockSpec(a.shape, lambda i: (0,) * a.ndim)
    out = pl.pallas_call(
        functools.partial(_fused_body, ndim=ndim, inter=inter),
        grid=(n_pad // blk,),
        in_specs=[pl.BlockSpec((blk, ndim), lambda i: (i, 0))]
                 + [full(a) for a in args],
        out_specs=pl.BlockSpec((blk, zdim), lambda i: (i, 0)),
        out_shape=jax.ShapeDtypeStruct((n_pad, zdim), jnp.float32),
        scratch_shapes=[pltpu.VMEM((ndim * inter, inter), jnp.float32),
                        pltpu.VMEM((1, inter), jnp.float32)],
    )(xp, *args)
    return out[:n] if n_pad != n else out
